# baseline (speedup 1.0000x reference)
import functools

import jax
import jax.numpy as jnp
from jax import lax
from jax.experimental import pallas as pl
from jax.experimental.pallas import tpu as pltpu
from jax.experimental.pallas import tpu_sc as plsc

_LOD = 128
_FEAT = 18

_NC = 2
_NS = 16
_NW = _NC * _NS


def _idx_body(p_ref, ptsT_ref, idx_ref):
    x = ptsT_ref[0:1, :]
    y = ptsT_ref[1:2, :]
    z = ptsT_ref[2:3, :]
    xt = p_ref[0] * x + p_ref[1] * y + p_ref[2] * z - p_ref[9]
    yt = p_ref[3] * x + p_ref[4] * y + p_ref[5] * z - p_ref[10]
    zt = p_ref[6] * x + p_ref[7] * y + p_ref[8] * z - p_ref[11]
    r = float(_LOD - 1)
    fx = jnp.floor(xt * r)
    fy = jnp.floor(yt * r)
    fz = jnp.floor(zt * r)
    idx_ref[...] = (fx + fy * float(_LOD) + fz * float(_LOD * _LOD)).astype(
        jnp.int32)


def _compute_idx(params, ptsT):
    n = ptsT.shape[1]
    bn = min(n, 131072)
    grid = n // bn
    return pl.pallas_call(
        _idx_body,
        grid=(grid,),
        in_specs=[
            pl.BlockSpec(memory_space=pltpu.SMEM),
            pl.BlockSpec((3, bn), lambda i: (0, i)),
        ],
        out_specs=pl.BlockSpec((1, bn), lambda i: (0, i)),
        out_shape=jax.ShapeDtypeStruct((1, n), jnp.int32),
    )(params, ptsT)


def _make_sc_gather(n, v, d, ch):
    bpw = n // _NW
    nchunk = bpw // ch
    mesh = plsc.VectorSubcoreMesh(
        core_axis_name="c", subcore_axis_name="s",
        num_cores=_NC, num_subcores=_NS)

    @functools.partial(
        pl.kernel,
        mesh=mesh,
        compiler_params=pltpu.CompilerParams(use_tc_tiling_on_sc=False),
        out_type=jax.ShapeDtypeStruct((n, d), jnp.float32),
        scratch_types=[
            pltpu.VMEM((ch,), jnp.int32),
            pltpu.VMEM((ch, d), jnp.float32),
            pltpu.SemaphoreType.DMA,
        ],
    )
    def sc_gather(idx_hbm, cb_hbm, out_hbm, idx_v, rows_v, sem):
        wid = lax.axis_index("s") * _NC + lax.axis_index("c")
        base = wid * bpw

        def chunk(i, carry):
            off = base + i * ch
            pltpu.sync_copy(idx_hbm.at[pl.ds(off, ch)], idx_v)
            pltpu.async_copy(cb_hbm.at[idx_v], rows_v, sem).wait()
            pltpu.sync_copy(rows_v, out_hbm.at[pl.ds(off, ch)])
            return carry

        lax.fori_loop(0, nchunk, chunk, 0)

    return sc_gather


def kernel(pts, transform, codebook_0):
    if pts.ndim > 2:
        pts = pts.reshape(-1, pts.shape[-1])
    n = pts.shape[0]
    a_inv = jnp.linalg.inv(transform[:3, :3])
    c = a_inv @ transform[:3, 3]
    params = jnp.concatenate([a_inv.reshape(-1), c]).astype(jnp.float32)
    idx = _compute_idx(params, pts.T).reshape(-1)
    gather = _make_sc_gather(n, codebook_0.shape[0], codebook_0.shape[1],
                             2048)
    return gather(idx, codebook_0)
